# Initial kernel scaffold; baseline (speedup 1.0000x reference)
import jax
import jax.numpy as jnp
from jax.experimental import pallas as pl


def kernel(x, edge_index, edge_attr, batch, We0, be0, W10, b10, g0, bt0, W20, b20, We1, be1, W11, b11, g1, bt1, W21, b21, We2, be2, W12, b12, g2, bt2, W22, b22):
    raise NotImplementedError("write your pallas kernel here")


# speedup vs baseline: 2.4769x; 2.4769x over previous
import functools

import jax
import jax.numpy as jnp
from jax import lax
from jax.experimental import pallas as pl
from jax.experimental.pallas import tpu as pltpu
from jax.experimental.pallas import tpu_sc as plsc

N = 10000
E = 320000
D = 128
ED = 16
G = 64

NC = 2
NS = 16
NW = NC * NS
EPW = E // NW
C = 80
NCH = EPW // C
NPAD = 10240
NPT = NPAD // NS
ZR = 128


EB = 2000


def _edge_mlp_body(ea, w0, b0, w1, b1, w2, b2, o0, o1, o2):
    a = ea[...]
    o0[...] = jnp.dot(a, w0[...], preferred_element_type=jnp.float32) + b0[...]
    o1[...] = jnp.dot(a, w1[...], preferred_element_type=jnp.float32) + b1[...]
    o2[...] = jnp.dot(a, w2[...], preferred_element_type=jnp.float32) + b2[...]


def _edge_mlp(edge_attr, We, be):
    wspec = pl.BlockSpec((ED, D), lambda i: (0, 0))
    bspec = pl.BlockSpec((1, D), lambda i: (0, 0))
    ospec = pl.BlockSpec((EB, D), lambda i: (i, 0))
    return pl.pallas_call(
        _edge_mlp_body,
        grid=(E // EB,),
        in_specs=[pl.BlockSpec((EB, ED), lambda i: (i, 0)),
                  wspec, bspec, wspec, bspec, wspec, bspec],
        out_specs=[ospec, ospec, ospec],
        out_shape=[jax.ShapeDtypeStruct((E, D), jnp.float32)] * 3,
    )(edge_attr, We[0], be[0], We[1], be[1], We[2], be[2])


def _sc_agg_body(x_hbm, e_hbm, src_hbm, dst_hbm, out_hbm,
                 idx_v, dst_v, xrows, erows, zrow, agg_sh, sem):
    c = lax.axis_index("c")
    s = lax.axis_index("s")
    wid = c * NS + s

    def zbody(r, _):
        for k in range(D // 16):
            zrow[r, pl.ds(k * 16, 16)] = jnp.zeros((16,), jnp.float32)
        return 0
    lax.fori_loop(0, ZR, zbody, 0)
    for j in range(NPT // ZR):
        pltpu.sync_copy(zrow, agg_sh.at[pl.ds(s * NPT + j * ZR, ZR)])
    plsc.subcore_barrier()

    base_e = wid * EPW

    def chunk(ch, _):
        base = base_e + ch * C
        pltpu.sync_copy(src_hbm.at[pl.ds(base, C)], idx_v)
        pltpu.sync_copy(dst_hbm.at[pl.ds(base, C)], dst_v)
        pltpu.sync_copy(e_hbm.at[pl.ds(base, C)], erows)
        pltpu.async_copy(x_hbm.at[idx_v], xrows, sem).wait()

        def row(r, _):
            for k in range(D // 16):
                sl = pl.ds(k * 16, 16)
                xrows[r, sl] = jnp.maximum(xrows[r, sl] + erows[r, sl], 0.0)
            return 0
        lax.fori_loop(0, C, row, 0)
        pltpu.sync_copy(xrows, agg_sh.at[dst_v], add=True)
        return 0
    lax.fori_loop(0, NCH, chunk, 0)

    plsc.subcore_barrier()
    pltpu.sync_copy(agg_sh.at[pl.ds(s * NPT, NPT)],
                    out_hbm.at[c, pl.ds(s * NPT, NPT)])


_sc_agg = functools.partial(
    pl.kernel,
    out_type=jax.ShapeDtypeStruct((NC, NPAD, D), jnp.float32),
    mesh=plsc.VectorSubcoreMesh(core_axis_name="c", subcore_axis_name="s"),
    scratch_types=[
        pltpu.VMEM((C,), jnp.int32),
        pltpu.VMEM((C,), jnp.int32),
        pltpu.VMEM((C, D), jnp.float32),
        pltpu.VMEM((C, D), jnp.float32),
        pltpu.VMEM((ZR, D), jnp.float32),
        pltpu.VMEM_SHARED((NPAD, D), jnp.float32),
        pltpu.SemaphoreType.DMA,
    ],
)(_sc_agg_body)


RB = 1000


def _lin1_body(x, a0, a1, w, b, t, stats):
    i = pl.program_id(0)

    @pl.when(i == 0)
    def _():
        stats[...] = jnp.zeros_like(stats)

    h = x[...] + a0[0] + a1[0]
    tv = jnp.dot(h, w[...], preferred_element_type=jnp.float32) + b[...]
    t[...] = tv
    stats[0:1, :] += jnp.sum(tv, axis=0, keepdims=True)
    stats[1:2, :] += jnp.sum(tv * tv, axis=0, keepdims=True)


def _lin1(x, agg, W1, b1):
    return pl.pallas_call(
        _lin1_body,
        grid=(N // RB,),
        in_specs=[pl.BlockSpec((RB, D), lambda i: (i, 0)),
                  pl.BlockSpec((1, RB, D), lambda i: (0, i, 0)),
                  pl.BlockSpec((1, RB, D), lambda i: (1, i, 0)),
                  pl.BlockSpec((D, D), lambda i: (0, 0)),
                  pl.BlockSpec((1, D), lambda i: (0, 0))],
        out_specs=[pl.BlockSpec((RB, D), lambda i: (i, 0)),
                   pl.BlockSpec((8, D), lambda i: (0, 0))],
        out_shape=[jax.ShapeDtypeStruct((N, D), jnp.float32),
                   jax.ShapeDtypeStruct((8, D), jnp.float32)],
    )(x, agg, agg, W1, b1.reshape(1, D))


def _lin2_body(t, sc, sh, w, b, out, *, final_relu):
    h = jnp.maximum(t[...] * sc[...] + sh[...], 0.0)
    o = jnp.dot(h, w[...], preferred_element_type=jnp.float32) + b[...]
    if final_relu:
        o = jnp.maximum(o, 0.0)
    out[...] = o


def _lin2(t, scale, shift, W2, b2, final_relu):
    return pl.pallas_call(
        functools.partial(_lin2_body, final_relu=final_relu),
        grid=(N // RB,),
        in_specs=[pl.BlockSpec((RB, D), lambda i: (i, 0)),
                  pl.BlockSpec((1, D), lambda i: (0, 0)),
                  pl.BlockSpec((1, D), lambda i: (0, 0)),
                  pl.BlockSpec((D, D), lambda i: (0, 0)),
                  pl.BlockSpec((1, D), lambda i: (0, 0))],
        out_specs=pl.BlockSpec((RB, D), lambda i: (i, 0)),
        out_shape=jax.ShapeDtypeStruct((N, D), jnp.float32),
    )(t, scale.reshape(1, D), shift.reshape(1, D), W2, b2.reshape(1, D))


def _pool_body(b3, h, out, acc, cnt):
    i = pl.program_id(0)
    nb = pl.num_programs(0)

    @pl.when(i == 0)
    def _():
        acc[...] = jnp.zeros_like(acc)
        cnt[...] = jnp.zeros_like(cnt)

    onehot = (lax.broadcasted_iota(jnp.int32, (G, RB), 0) == b3[0]).astype(
        jnp.float32)
    acc[...] += jnp.dot(onehot, h[...], preferred_element_type=jnp.float32)
    cnt[...] += jnp.broadcast_to(jnp.sum(onehot, axis=1, keepdims=True), (G, D))

    @pl.when(i == nb - 1)
    def _():
        out[...] = acc[...] / jnp.maximum(cnt[...], 1.0)


def _pool(h, batch):
    b3 = batch.reshape(N // RB, 1, RB)
    return pl.pallas_call(
        _pool_body,
        grid=(N // RB,),
        in_specs=[pl.BlockSpec((1, 1, RB), lambda i: (i, 0, 0)),
                  pl.BlockSpec((RB, D), lambda i: (i, 0))],
        out_specs=pl.BlockSpec((G, D), lambda i: (0, 0)),
        out_shape=jax.ShapeDtypeStruct((G, D), jnp.float32),
        scratch_shapes=[pltpu.VMEM((G, D), jnp.float32),
                        pltpu.VMEM((G, D), jnp.float32)],
    )(b3, h)


def kernel(x, edge_index, edge_attr, batch,
           We0, be0, W10, b10, g0, bt0, W20, b20,
           We1, be1, W11, b11, g1, bt1, W21, b21,
           We2, be2, W12, b12, g2, bt2, W22, b22):
    src = edge_index[0]
    dst = edge_index[1]
    es = _edge_mlp(edge_attr, (We0, We1, We2),
                   (be0.reshape(1, D), be1.reshape(1, D), be2.reshape(1, D)))
    params = ((W10, b10, g0, bt0, W20, b20),
              (W11, b11, g1, bt1, W21, b21),
              (W12, b12, g2, bt2, W22, b22))
    h = x
    for i in range(3):
        W1, b1, g, bt, W2, b2 = params[i]
        agg = _sc_agg(h, es[i], src, dst)
        t, stats = _lin1(h, agg, W1, b1)
        mu = stats[0] / N
        var = stats[1] / N - mu * mu
        scale = g * lax.rsqrt(var + 1e-5)
        shift = bt - mu * scale
        h = _lin2(t, scale, shift, W2, b2, final_relu=(i != 2))
    return _pool(h, batch)
